# baseline (speedup 1.0000x reference)
import functools

import jax
import jax.numpy as jnp
from jax import lax
from jax.experimental import pallas as pl
from jax.experimental.pallas import tpu as pltpu
from jax.experimental.pallas import tpu_sc as plsc

B, L, H = 1024, 200, 128
LP1 = L + 1
LPAD = 208
NC, NS = 2, 16
NW = NC * NS
BH = B // 2
NB = BH // NW
NL = H // 16
EPS = 1e-12
RU = 5
RSQRT_MAGIC = 0x5F3759DF


def _rsqrt16(a):
    i = plsc.bitcast(a, jnp.int32)
    i = jnp.full((16,), RSQRT_MAGIC, dtype=jnp.int32) - lax.shift_right_logical(i, 1)
    y = plsc.bitcast(i, jnp.float32)
    half_a = a * 0.5
    for _ in range(2):
        y = y * (1.5 - half_a * y * y)
    return y


def _make_half_kernel(half):
    gbase0 = half * BH

    @functools.partial(
        pl.kernel,
        out_type=jax.ShapeDtypeStruct((BH, LP1, H), jnp.float32),
        mesh=plsc.VectorSubcoreMesh(core_axis_name="c", subcore_axis_name="s"),
        compiler_params=pltpu.CompilerParams(
            use_tc_tiling_on_sc=True, needs_layout_passes=False),
        scratch_types=[
            pltpu.VMEM((NB * L,), jnp.int32),
            pltpu.VMEM((NB, H), jnp.float32),
            pltpu.VMEM((2, LPAD, H), jnp.float32),
            pltpu.VMEM((LPAD, H), jnp.float32),
            pltpu.VMEM((H,), jnp.float32),
            pltpu.VMEM((H,), jnp.float32),
            pltpu.SemaphoreType.DMA,
            pltpu.SemaphoreType.DMA,
            pltpu.SemaphoreType.DMA,
            pltpu.SemaphoreType.DMA,
        ],
    )
    def _emb_ln_kernel(ids_hbm, vec_hbm, wt_hbm, pt_hbm, g_hbm, b_hbm,
                       out_hbm, idx_v, vecs_v, rows_v, pos_v, g_v, b_v,
                       gsem0, gsem1, osem0, osem1):
        wid = lax.axis_index("s") * NC + lax.axis_index("c")
        base = wid * NB
        gb = gbase0 + base
        gsem = (gsem0, gsem1)
        osem = (osem0, osem1)

        pltpu.sync_copy(ids_hbm.at[pl.ds(pl.multiple_of(gb * L, 8), NB * L)], idx_v)
        pltpu.sync_copy(vec_hbm.at[pl.ds(pl.multiple_of(gb, 8), NB)], vecs_v)
        pltpu.sync_copy(pt_hbm.at[pl.ds(0, LPAD)], pos_v)
        pltpu.sync_copy(g_hbm, g_v)
        pltpu.sync_copy(b_hbm, b_v)

        gamma = [g_v[pl.ds(c * 16, 16)] for c in range(NL)]
        beta = [b_v[pl.ds(c * 16, 16)] for c in range(NL)]

        def gather_copies(p, i, make_only=False):
            mk = pltpu.make_async_copy if make_only else pltpu.async_copy
            c0 = mk(wt_hbm.at[idx_v.at[pl.ds(pl.multiple_of(i * L, 8), 104)]],
                    rows_v.at[p, pl.ds(0, 104)], gsem[p])
            c1 = mk(wt_hbm.at[idx_v.at[pl.ds(pl.multiple_of(i * L + 104, 8), 96)]],
                    rows_v.at[p, pl.ds(104, 96)], gsem[p])
            return c0, c1

        def wait_gather(p, i):
            for c in gather_copies(p, i, make_only=True):
                c.wait()

        def ln8(x):
            s = ((x[0] + x[1]) + (x[2] + x[3])) + ((x[4] + x[5]) + (x[6] + x[7]))
            sq = [xc * xc for xc in x]
            q = ((sq[0] + sq[1]) + (sq[2] + sq[3])) + ((sq[4] + sq[5]) + (sq[6] + sq[7]))
            tot = jnp.full((16,), jnp.sum(s))
            tot2 = jnp.full((16,), jnp.sum(q))
            mean = tot * (1.0 / H)
            var = tot2 * (1.0 / H) - mean * mean
            inv = _rsqrt16(var + EPS)
            return [(x[c] - mean) * inv * gamma[c] + beta[c] for c in range(NL)]

        def compute(p, i):
            def row_body(j, c2):
                for u in range(RU):
                    l = j * RU + u
                    x = [rows_v[p, l, pl.ds(c * 16, 16)]
                         + pos_v[l + 1, pl.ds(c * 16, 16)] for c in range(NL)]
                    o = ln8(x)
                    for c in range(NL):
                        rows_v[p, l, pl.ds(c * 16, 16)] = o[c]
                return c2

            lax.fori_loop(0, L // RU, row_body, 0, unroll=False)
            x = [vecs_v[i, pl.ds(c * 16, 16)] + pos_v[L + 1, pl.ds(c * 16, 16)]
                 for c in range(NL)]
            o = ln8(x)
            for c in range(NL):
                rows_v[p, L, pl.ds(c * 16, 16)] = o[c]

        gather_copies(0, 0)

        def pair_body(j, carry):
            i0 = 2 * j
            i1 = i0 + 1

            @pl.when(j > 0)
            def _():
                pltpu.make_async_copy(rows_v.at[1, pl.ds(0, LP1)],
                                      out_hbm.at[base + i0 - 1], osem[1]).wait()

            gather_copies(1, i1)
            wait_gather(0, i0)
            compute(0, i0)
            out0 = pltpu.async_copy(rows_v.at[0, pl.ds(0, LP1)],
                                    out_hbm.at[base + i0], osem[0])
            wait_gather(1, i1)
            out0.wait()

            @pl.when(j < NB // 2 - 1)
            def _():
                gather_copies(0, i0 + 2)
            compute(1, i1)
            pltpu.async_copy(rows_v.at[1, pl.ds(0, LP1)],
                             out_hbm.at[base + i1], osem[1])
            return carry

        lax.fori_loop(0, NB // 2, pair_body, 0, unroll=False)
        pltpu.make_async_copy(rows_v.at[1, pl.ds(0, LP1)],
                              out_hbm.at[base + NB - 1], osem[1]).wait()

    return _emb_ln_kernel


_half_kernels = (_make_half_kernel(0), _make_half_kernel(1))


def kernel(input_ids, vectors, word_table, pos_table, gamma, beta):
    ids_flat = input_ids.astype(jnp.int32).reshape(B * L)
    o0 = _half_kernels[0](ids_flat, vectors, word_table, pos_table, gamma, beta)
    o1 = _half_kernels[1](ids_flat, vectors, word_table, pos_table, gamma, beta)
    out = jnp.zeros((B, LP1, H), dtype=jnp.float32)
    out = lax.dynamic_update_slice(out, o0, (0, 0, 0))
    out = lax.dynamic_update_slice(out, o1, (BH, 0, 0))
    return out


# speedup vs baseline: 1.5150x; 1.5150x over previous
import functools

import jax
import jax.numpy as jnp
from jax import lax
from jax.experimental import pallas as pl
from jax.experimental.pallas import tpu as pltpu
from jax.experimental.pallas import tpu_sc as plsc

B, L, H = 1024, 200, 128
LP1 = L + 1
LPAD = 208
NC, NS = 2, 16
NW = NC * NS
NB = B // NW
NL = H // 16
EPS = 1e-12
RU = 5
RSQRT_MAGIC = 0x5F3759DF


def _rsqrt16(a):
    i = plsc.bitcast(a, jnp.int32)
    i = jnp.full((16,), RSQRT_MAGIC, dtype=jnp.int32) - lax.shift_right_logical(i, 1)
    y = plsc.bitcast(i, jnp.float32)
    half_a = a * 0.5
    for _ in range(2):
        y = y * (1.5 - half_a * y * y)
    return y


@functools.partial(
    pl.kernel,
    out_type=jax.ShapeDtypeStruct((B, LP1, H), jnp.float32),
    mesh=plsc.VectorSubcoreMesh(core_axis_name="c", subcore_axis_name="s"),
    compiler_params=pltpu.CompilerParams(
        use_tc_tiling_on_sc=True, needs_layout_passes=False),
    scratch_types=[
        pltpu.VMEM((NB * L,), jnp.int32),
        pltpu.VMEM((NB, H), jnp.float32),
        pltpu.VMEM((3, LPAD, H), jnp.float32),
        pltpu.VMEM((LPAD, H), jnp.float32),
        pltpu.VMEM((H,), jnp.float32),
        pltpu.VMEM((H,), jnp.float32),
        pltpu.SemaphoreType.DMA,
        pltpu.SemaphoreType.DMA,
        pltpu.SemaphoreType.DMA,
        pltpu.SemaphoreType.DMA,
        pltpu.SemaphoreType.DMA,
        pltpu.SemaphoreType.DMA,
    ],
)
def _emb_ln_kernel(ids_hbm, vec_hbm, wt_hbm, pt_hbm, g_hbm, b_hbm,
                   out_hbm, idx_v, vecs_v, rows_v, pos_v, g_v, b_v,
                   gsem0, gsem1, gsem2, osem0, osem1, osem2):
    wid = lax.axis_index("s") * NC + lax.axis_index("c")
    base = wid * NB
    gsem = (gsem0, gsem1, gsem2)
    osem = (osem0, osem1, osem2)

    pltpu.sync_copy(ids_hbm.at[pl.ds(pl.multiple_of(base * L, 8), NB * L)], idx_v)
    pltpu.sync_copy(vec_hbm.at[pl.ds(base, NB)], vecs_v)
    pltpu.sync_copy(pt_hbm.at[pl.ds(0, LPAD)], pos_v)
    pltpu.sync_copy(g_hbm, g_v)
    pltpu.sync_copy(b_hbm, b_v)

    gamma = [g_v[pl.ds(c * 16, 16)] for c in range(NL)]
    beta = [b_v[pl.ds(c * 16, 16)] for c in range(NL)]

    def gather_copies(p, i, make_only=False):
        mk = pltpu.make_async_copy if make_only else pltpu.async_copy
        c0 = mk(wt_hbm.at[idx_v.at[pl.ds(pl.multiple_of(i * L, 8), 104)]],
                rows_v.at[p, pl.ds(0, 104)], gsem[p])
        c1 = mk(wt_hbm.at[idx_v.at[pl.ds(pl.multiple_of(i * L + 104, 8), 96)]],
                rows_v.at[p, pl.ds(104, 96)], gsem[p])
        return c0, c1

    def wait_gather(p, i):
        for c in gather_copies(p, i, make_only=True):
            c.wait()

    def ln8(x):
        s = ((x[0] + x[1]) + (x[2] + x[3])) + ((x[4] + x[5]) + (x[6] + x[7]))
        sq = [xc * xc for xc in x]
        q = ((sq[0] + sq[1]) + (sq[2] + sq[3])) + ((sq[4] + sq[5]) + (sq[6] + sq[7]))
        tot = jnp.full((16,), jnp.sum(s))
        tot2 = jnp.full((16,), jnp.sum(q))
        mean = tot * (1.0 / H)
        var = tot2 * (1.0 / H) - mean * mean
        inv = _rsqrt16(var + EPS)
        return [(x[c] - mean) * inv * gamma[c] + beta[c] for c in range(NL)]

    def compute(p, i):
        def row_body(j, c2):
            for u in range(RU):
                l = j * RU + u
                x = [rows_v[p, l, pl.ds(c * 16, 16)] + pos_v[l + 1, pl.ds(c * 16, 16)]
                     for c in range(NL)]
                o = ln8(x)
                for c in range(NL):
                    rows_v[p, l, pl.ds(c * 16, 16)] = o[c]
            return c2

        lax.fori_loop(0, L // RU, row_body, 0, unroll=False)
        x = [vecs_v[i, pl.ds(c * 16, 16)] + pos_v[L + 1, pl.ds(c * 16, 16)]
             for c in range(NL)]
        o = ln8(x)
        for c in range(NL):
            rows_v[p, L, pl.ds(c * 16, 16)] = o[c]

    def out_copy(s, i, make_only=False):
        mk = pltpu.make_async_copy if make_only else pltpu.async_copy
        return mk(rows_v.at[s, pl.ds(0, LP1)], out_hbm.at[base + i], osem[s])

    def step(s, k):
        sn = (s + 1) % 3
        wait_gather(s, k)

        @pl.when(k >= 2)
        def _():
            out_copy(sn, k - 2, make_only=True).wait()

        @pl.when(k + 1 < NB)
        def _():
            gather_copies(sn, k + 1)
        compute(s, k)
        out_copy(s, k)

    gather_copies(0, 0)

    def ring_body(j, carry):
        for u in range(3):
            step(u, 3 * j + u)
        return carry

    lax.fori_loop(0, NB // 3, ring_body, 0, unroll=False)
    step(0, NB - 2)
    step(1, NB - 1)
    for t in range(NB - 2, NB):
        out_copy(t % 3, t, make_only=True).wait()


def kernel(input_ids, vectors, word_table, pos_table, gamma, beta):
    return _emb_ln_kernel(input_ids.astype(jnp.int32).reshape(B * L), vectors,
                          word_table, pos_table, gamma, beta)
